# Initial kernel scaffold; baseline (speedup 1.0000x reference)
import jax
import jax.numpy as jnp
from jax.experimental import pallas as pl


def kernel(x, edge_index, eps, mlp_w1, mlp_b1, mlp_w2, mlp_b2, bn_gamma, bn_beta, pred_w, pred_b):
    raise NotImplementedError("write your pallas kernel here")


# speedup vs baseline: 4.4898x; 4.4898x over previous
import functools

import jax
import jax.numpy as jnp
from jax import lax
from jax.experimental import pallas as pl
from jax.experimental.pallas import tpu as pltpu
from jax.experimental.pallas import tpu_sc as plsc

NC = 2
NS = 16
NW = NC * NS


def _sc_scatter_add(h, src, dst, n, d, e):
    e_per_w = e // NW
    chunk = 80
    n_chunks = e_per_w // chunk
    npad = ((n + NS * 8 - 1) // (NS * 8)) * (NS * 8)
    rows_per_tile = npad // NS
    zrows = 128
    n_zcopies = rows_per_tile // zrows

    mesh = plsc.VectorSubcoreMesh(core_axis_name="c", subcore_axis_name="s")

    @functools.partial(
        pl.kernel,
        out_type=jax.ShapeDtypeStruct((NC * npad, d), jnp.float32),
        mesh=mesh,
        scratch_types=[
            pltpu.VMEM((chunk,), jnp.int32),
            pltpu.VMEM((chunk,), jnp.int32),
            pltpu.VMEM((chunk, d), jnp.float32),
            pltpu.VMEM((zrows, d), jnp.float32),
            pltpu.VMEM_SHARED((npad, d), jnp.float32),
            pltpu.SemaphoreType.DMA,
        ],
    )
    def k(h_hbm, src_hbm, dst_hbm, out_hbm, sidx, didx, rows, zbuf, acc, sem):
        c = lax.axis_index("c")
        s = lax.axis_index("s")
        wid = c * NS + s

        zero16 = jnp.zeros((16,), jnp.float32)

        @pl.loop(0, zrows * (d // 16))
        def _(i):
            r = i // (d // 16)
            j = (i % (d // 16)) * 16
            zbuf[r, pl.ds(j, 16)] = zero16

        @pl.loop(0, n_zcopies)
        def _(t):
            pltpu.sync_copy(zbuf, acc.at[pl.ds(s * rows_per_tile + t * zrows, zrows)])

        plsc.subcore_barrier()

        ebase = wid * e_per_w

        @pl.loop(0, n_chunks)
        def _(kk):
            off = ebase + kk * chunk
            pltpu.sync_copy(src_hbm.at[pl.ds(off, chunk)], sidx)
            pltpu.sync_copy(dst_hbm.at[pl.ds(off, chunk)], didx)
            pltpu.async_copy(h_hbm.at[sidx], rows, sem).wait()
            pltpu.sync_copy(rows, acc.at[didx], add=True)

        plsc.subcore_barrier()

        rbase = s * rows_per_tile
        pltpu.sync_copy(
            acc.at[pl.ds(rbase, rows_per_tile)],
            out_hbm.at[pl.ds(c * npad + rbase, rows_per_tile)],
        )

    return k(h, src, dst), npad


def _tc_layer(part, h, scale, w1, b1, w2, b2, gamma, beta, n, d, npad):
    def body(part_ref, h_ref, scale_ref, w1_ref, b1_ref, w2_ref, b2_ref,
             g_ref, be_ref, out_ref, cs_ref):
        pooled = (part_ref[pl.ds(0, n), :] + part_ref[pl.ds(npad, n), :]
                  + scale_ref[...] * h_ref[...])
        hmid = jnp.dot(pooled, w1_ref[...], preferred_element_type=jnp.float32)
        hmid = jnp.maximum(hmid + b1_ref[...], 0.0)
        rep = jnp.dot(hmid, w2_ref[...], preferred_element_type=jnp.float32)
        rep = rep + b2_ref[...]
        m = jnp.mean(rep, axis=0, keepdims=True)
        v = jnp.mean((rep - m) * (rep - m), axis=0, keepdims=True)
        hn = (rep - m) * jax.lax.rsqrt(v + 1e-5) * g_ref[...] + be_ref[...]
        hn = jnp.maximum(hn, 0.0)
        out_ref[...] = hn
        cs_ref[...] = jnp.sum(hn, axis=0, keepdims=True)

    return pl.pallas_call(
        body,
        out_shape=[
            jax.ShapeDtypeStruct((n, d), jnp.float32),
            jax.ShapeDtypeStruct((1, d), jnp.float32),
        ],
    )(part, h, scale, w1, b1, w2, b2, gamma, beta)


def _tc_readout(x, colsums, pred_w, pred_b, n, d, num_layers):
    def body(x_ref, cs_ref, pw_ref, pb_ref, out_ref):
        acc = jnp.dot(jnp.sum(x_ref[...], axis=0, keepdims=True), pw_ref[0],
                      preferred_element_type=jnp.float32)
        for l in range(1, num_layers):
            acc = acc + jnp.dot(cs_ref[pl.ds(l - 1, 1), :], pw_ref[l],
                                preferred_element_type=jnp.float32)
        acc = acc + jnp.sum(pb_ref[...], axis=0, keepdims=True)
        out_ref[...] = acc

    return pl.pallas_call(
        body,
        out_shape=jax.ShapeDtypeStruct((1, d), jnp.float32),
    )(x, colsums, pred_w, pred_b)


def kernel(x, edge_index, eps, mlp_w1, mlp_b1, mlp_w2, mlp_b2,
           bn_gamma, bn_beta, pred_w, pred_b):
    n, d = x.shape
    e = edge_index.shape[1]
    num_layers = pred_w.shape[0]
    L = eps.shape[0]

    src = edge_index[0]
    dst = edge_index[1]

    h = x
    colsums = []
    for layer in range(L):
        part, npad = _sc_scatter_add(h, src, dst, n, d, e)
        scale = (1.0 + eps[layer]).reshape(1, 1)
        h, cs = _tc_layer(
            part, h, scale,
            mlp_w1[layer], mlp_b1[layer].reshape(1, d),
            mlp_w2[layer], mlp_b2[layer].reshape(1, d),
            bn_gamma[layer].reshape(1, d), bn_beta[layer].reshape(1, d),
            n, d, npad,
        )
        colsums.append(cs)

    cs_all = jnp.concatenate(colsums, axis=0)
    return _tc_readout(x, cs_all, pred_w, pred_b, n, d, num_layers)
